# baseline (speedup 1.0000x reference)
import functools

import jax
import jax.numpy as jnp
from jax import lax
from jax.experimental import pallas as pl
from jax.experimental.pallas import tpu as pltpu
from jax.experimental.pallas import tpu_sc as plsc

N = 10000
E = 320000
D = 128
F = 16
B = 64

NC = 2
NS = 16
NW = NC * NS
N_PAD = 10240
RPT = N_PAD // NS
EPT = E // NW
CHUNK = 2000
NCHUNK = EPT // CHUNK

PK = 128 // F
NPR = N // PK
NPR_PAD = N_PAD // PK


def _mesh():
    return plsc.VectorSubcoreMesh(
        core_axis_name="c", subcore_axis_name="s", num_cores=NC, num_subcores=NS
    )


@functools.cache
def _build_sc_degree():
    return pl.kernel(
        _sc_degree_body,
        out_type=jax.ShapeDtypeStruct((NC, N_PAD, F), jnp.float32),
        mesh=_mesh(),
        compiler_params=pltpu.CompilerParams(use_tc_tiling_on_sc=False),
        scratch_types=[
            pltpu.VMEM((EPT,), jnp.int32),
            pltpu.VMEM((EPT,), jnp.float32),
            pltpu.VMEM((RPT,), jnp.float32),
            pltpu.VMEM((RPT, F), jnp.float32),
            pltpu.VMEM_SHARED((N_PAD,), jnp.float32),
            pltpu.SemaphoreType.DMA,
        ],
    )


def _sc_degree_body(dst_hbm, zeros1_hbm, ones1_hbm, out_hbm, didx,
                    ones, dloc, zbuf, acc, sem):
    c = lax.axis_index("c")
    s = lax.axis_index("s")
    base = (s * NC + c) * EPT
    idxcp = pltpu.async_copy(dst_hbm.at[pl.ds(base, EPT)], didx, sem)
    pltpu.sync_copy(zeros1_hbm, dloc)
    pltpu.sync_copy(ones1_hbm, ones)
    pltpu.sync_copy(dloc, acc.at[pl.ds(s * RPT, RPT)])
    plsc.subcore_barrier()

    idxcp.wait()
    pltpu.sync_copy(ones, acc.at[didx], add=True)

    plsc.subcore_barrier()
    pltpu.sync_copy(acc.at[pl.ds(s * RPT, RPT)], dloc)

    def expand(g, carry):
        v = dloc[pl.ds(g * 16, 16)]
        for u in range(16):
            zbuf[g * 16 + u, :] = jnp.full((F,), v[u], jnp.float32)
        return carry

    lax.fori_loop(0, RPT // 16, expand, 0)
    pltpu.sync_copy(zbuf, out_hbm.at[c, pl.ds(s * RPT, RPT)])


@functools.cache
def _build_sc_aggregate():
    return pl.kernel(
        _sc_aggregate_body,
        out_type=jax.ShapeDtypeStruct((NC, N_PAD, F), jnp.float32),
        mesh=_mesh(),
        compiler_params=pltpu.CompilerParams(use_tc_tiling_on_sc=False),
        scratch_types=[
            pltpu.VMEM((CHUNK,), jnp.int32),
            pltpu.VMEM((CHUNK,), jnp.int32),
            pltpu.VMEM((CHUNK,), jnp.int32),
            pltpu.VMEM((CHUNK,), jnp.int32),
            pltpu.VMEM((CHUNK, F), jnp.float32),
            pltpu.VMEM((CHUNK, F), jnp.float32),
            pltpu.VMEM((RPT, F), jnp.float32),
            pltpu.VMEM_SHARED((N_PAD, F), jnp.float32),
            pltpu.SemaphoreType.DMA,
            pltpu.SemaphoreType.DMA,
            pltpu.SemaphoreType.DMA,
            pltpu.SemaphoreType.DMA,
        ],
    )


def _sc_aggregate_body(y_hbm, ei_hbm, zeros_hbm, out_hbm,
                       sidx0, sidx1, didx0, didx1, rows0, rows1, zbuf, acc,
                       gsem0, gsem1, isem0, isem1):
    c = lax.axis_index("c")
    s = lax.axis_index("s")
    base = (s * NC + c) * EPT
    sbuf = (sidx0, sidx1)
    dbuf = (didx0, didx1)
    rbuf = (rows0, rows1)
    gsems = (gsem0, gsem1)
    isems = (isem0, isem1)

    def idx_start(k):
        b = k % 2
        e0 = base + k * CHUNK
        cs = pltpu.async_copy(ei_hbm.at[0, pl.ds(e0, CHUNK)], sbuf[b],
                              isems[b])
        cd = pltpu.async_copy(ei_hbm.at[1, pl.ds(e0, CHUNK)], dbuf[b],
                              isems[b])
        return (cs, cd)

    idx = [None, None]
    gathers = [None, None]
    idx[0] = idx_start(0)
    if NCHUNK > 1:
        idx[1] = idx_start(1)
    idx[0][0].wait()
    idx[0][1].wait()
    gathers[0] = pltpu.async_copy(y_hbm.at[sbuf[0]], rows0, gsem0)

    pltpu.sync_copy(zeros_hbm, zbuf)
    pltpu.sync_copy(zbuf, acc.at[pl.ds(s * RPT, RPT)])
    plsc.subcore_barrier()

    for k in range(NCHUNK):
        b = k % 2
        nb = (k + 1) % 2
        if k + 1 < NCHUNK:
            idx[nb][0].wait()
            idx[nb][1].wait()
            gathers[nb] = pltpu.async_copy(y_hbm.at[sbuf[nb]], rbuf[nb],
                                           gsems[nb])
        gathers[b].wait()
        pltpu.sync_copy(rbuf[b], acc.at[dbuf[b]], add=True)
        if k + 2 < NCHUNK:
            idx[b] = idx_start(k + 2)

    plsc.subcore_barrier()
    pltpu.sync_copy(acc.at[pl.ds(s * RPT, RPT)], out_hbm.at[c, pl.ds(s * RPT, RPT)])


def _tc_mm1_body(x_ref, w_ref, xw_ref):
    xw_ref[...] = jnp.dot(x_ref[...], w_ref[...],
                          preferred_element_type=jnp.float32)


def _dinv_packed(deg_ref):
    return lax.rsqrt(deg_ref[0, :NPR] + deg_ref[1, :NPR] + 1.0)


def _tc_scale_body(xw_ref, deg_ref, y_ref):
    y_ref[...] = xw_ref[...] * _dinv_packed(deg_ref)


def _tc_mid_body(acc_ref, y_ref, deg_ref, b_ref, wbd_ref, out_ref):
    dinv = _dinv_packed(deg_ref)
    h = dinv * (acc_ref[0, :NPR] + acc_ref[1, :NPR] + y_ref[...]) + b_ref[...]
    h = jnp.maximum(h, 0.0)
    out_ref[...] = jnp.dot(h, wbd_ref[...],
                           preferred_element_type=jnp.float32) * dinv


def _tc_final_body(acc_ref, y_ref, deg_ref, b_ref, batcht_ref, out_ref):
    dinv = _dinv_packed(deg_ref)
    z = dinv * (acc_ref[0, :NPR] + acc_ref[1, :NPR] + y_ref[...]) + b_ref[...]
    z = jnp.maximum(z, 0.0)
    gid = lax.broadcasted_iota(jnp.int32, (B, 1), 0)
    psum = jnp.zeros((B, F), jnp.float32)
    cnt = jnp.zeros((B, 1), jnp.float32)
    for u in range(PK):
        oh = (batcht_ref[u:u + 1, :] == gid).astype(jnp.float32)
        psum = psum + jnp.dot(oh, z[:, u * F:(u + 1) * F],
                              preferred_element_type=jnp.float32)
        cnt = cnt + jnp.sum(oh, axis=1, keepdims=True)
    pooled = psum / jnp.maximum(cnt, 1.0)
    m = jnp.max(pooled, axis=1, keepdims=True)
    ex = jnp.exp(pooled - m)
    lse = jnp.log(jnp.sum(ex, axis=1, keepdims=True))
    out_ref[...] = pooled - m - lse


_tc_mm1 = pl.pallas_call(
    _tc_mm1_body, out_shape=jax.ShapeDtypeStruct((NPR, 128), jnp.float32))
_tc_scale = pl.pallas_call(
    _tc_scale_body, out_shape=jax.ShapeDtypeStruct((NPR, 128), jnp.float32))
_tc_mid = pl.pallas_call(
    _tc_mid_body, out_shape=jax.ShapeDtypeStruct((NPR, 128), jnp.float32))
_tc_final = pl.pallas_call(
    _tc_final_body, out_shape=jax.ShapeDtypeStruct((B, F), jnp.float32))


def kernel(x, edge_index, batch, W1, b1, W2, b2):
    zeros_rp = jnp.zeros((RPT, F), jnp.float32)
    zeros1 = jnp.zeros((RPT,), jnp.float32)
    ones1 = jnp.ones((EPT,), jnp.float32)
    w1bd = jnp.kron(jnp.eye(PK, dtype=jnp.float32), W1)
    w2bd = jnp.kron(jnp.eye(PK, dtype=jnp.float32), W2)
    b1t = jnp.tile(b1, PK).reshape(1, 128)
    b2t = jnp.tile(b2, PK).reshape(1, 128)
    batch_t = batch.reshape(NPR, PK).T

    sc_degree = _build_sc_degree()
    sc_aggregate = _build_sc_aggregate()

    deg16 = sc_degree(edge_index[1], zeros1, ones1)
    degp = deg16.reshape(NC, NPR_PAD, 128)
    xwp = _tc_mm1(x.reshape(NPR, PK * D), w1bd)
    y1p = _tc_scale(xwp, degp)
    acc1 = sc_aggregate(y1p.reshape(N, F), edge_index, zeros_rp)
    y2p = _tc_mid(acc1.reshape(NC, NPR_PAD, 128), y1p, degp, b1t, w2bd)
    acc2 = sc_aggregate(y2p.reshape(N, F), edge_index, zeros_rp)
    out = _tc_final(acc2.reshape(NC, NPR_PAD, 128), y2p, degp, b2t, batch_t)
    return out


# speedup vs baseline: 1.1866x; 1.1866x over previous
import functools

import jax
import jax.numpy as jnp
from jax import lax
from jax.experimental import pallas as pl
from jax.experimental.pallas import tpu as pltpu
from jax.experimental.pallas import tpu_sc as plsc

N = 10000
E = 320000
D = 128
F = 16
B = 64

NC = 2
NS = 16
NW = NC * NS
N_PAD = 10240
RPT = N_PAD // NS
EPT = E // NW
CHUNK = 2000
NCHUNK = EPT // CHUNK

PK = 128 // F
NPR = N // PK
NPR_PAD = N_PAD // PK


def _mesh():
    return plsc.VectorSubcoreMesh(
        core_axis_name="c", subcore_axis_name="s", num_cores=NC, num_subcores=NS
    )


@functools.cache
def _build_sc_degree():
    return pl.kernel(
        _sc_degree_body,
        out_type=jax.ShapeDtypeStruct((NC, N_PAD, F), jnp.float32),
        mesh=_mesh(),
        compiler_params=pltpu.CompilerParams(use_tc_tiling_on_sc=False),
        scratch_types=[
            pltpu.VMEM((EPT,), jnp.int32),
            pltpu.VMEM((EPT,), jnp.float32),
            pltpu.VMEM((RPT,), jnp.float32),
            pltpu.VMEM((RPT, F), jnp.float32),
            pltpu.VMEM_SHARED((N_PAD,), jnp.float32),
            pltpu.SemaphoreType.DMA,
        ],
    )


def _sc_degree_body(ei_hbm, zeros1_hbm, ones1_hbm, out_hbm, didx,
                    ones, dloc, zbuf, acc, sem):
    c = lax.axis_index("c")
    s = lax.axis_index("s")
    base = (s * NC + c) * EPT
    idxcp = pltpu.async_copy(ei_hbm.at[1, pl.ds(base, EPT)], didx, sem)
    pltpu.sync_copy(zeros1_hbm, dloc)
    pltpu.sync_copy(ones1_hbm, ones)
    pltpu.sync_copy(dloc, acc.at[pl.ds(s * RPT, RPT)])
    plsc.subcore_barrier()

    idxcp.wait()
    pltpu.sync_copy(ones, acc.at[didx], add=True)

    plsc.subcore_barrier()
    pltpu.sync_copy(acc.at[pl.ds(s * RPT, RPT)], dloc)

    def expand(g, carry):
        v = dloc[pl.ds(g * 16, 16)]
        for u in range(16):
            zbuf[g * 16 + u, :] = jnp.full((F,), v[u], jnp.float32)
        return carry

    lax.fori_loop(0, RPT // 16, expand, 0)
    pltpu.sync_copy(zbuf, out_hbm.at[c, pl.ds(s * RPT, RPT)])


@functools.cache
def _build_sc_aggregate():
    return pl.kernel(
        _sc_aggregate_body,
        out_type=jax.ShapeDtypeStruct((NC, N_PAD, F), jnp.float32),
        mesh=_mesh(),
        compiler_params=pltpu.CompilerParams(use_tc_tiling_on_sc=False),
        scratch_types=[
            pltpu.VMEM((CHUNK,), jnp.int32),
            pltpu.VMEM((CHUNK,), jnp.int32),
            pltpu.VMEM((CHUNK,), jnp.int32),
            pltpu.VMEM((CHUNK,), jnp.int32),
            pltpu.VMEM((CHUNK, F), jnp.float32),
            pltpu.VMEM((CHUNK, F), jnp.float32),
            pltpu.VMEM((RPT, F), jnp.float32),
            pltpu.VMEM_SHARED((N_PAD, F), jnp.float32),
            pltpu.SemaphoreType.DMA,
            pltpu.SemaphoreType.DMA,
            pltpu.SemaphoreType.DMA,
            pltpu.SemaphoreType.DMA,
        ],
    )


def _sc_aggregate_body(y_hbm, ei_hbm, zeros_hbm, out_hbm,
                       sidx0, sidx1, didx0, didx1, rows0, rows1, zbuf, acc,
                       gsem0, gsem1, isem0, isem1):
    c = lax.axis_index("c")
    s = lax.axis_index("s")
    base = (s * NC + c) * EPT
    sbuf = (sidx0, sidx1)
    dbuf = (didx0, didx1)
    rbuf = (rows0, rows1)
    gsems = (gsem0, gsem1)
    isems = (isem0, isem1)

    def idx_start(k):
        b = k % 2
        e0 = base + k * CHUNK
        cs = pltpu.async_copy(ei_hbm.at[0, pl.ds(e0, CHUNK)], sbuf[b],
                              isems[b])
        cd = pltpu.async_copy(ei_hbm.at[1, pl.ds(e0, CHUNK)], dbuf[b],
                              isems[b])
        return (cs, cd)

    idx = [None, None]
    gathers = [None, None]
    idx[0] = idx_start(0)
    if NCHUNK > 1:
        idx[1] = idx_start(1)
    idx[0][0].wait()
    idx[0][1].wait()
    gathers[0] = pltpu.async_copy(y_hbm.at[sbuf[0]], rows0, gsem0)

    pltpu.sync_copy(zeros_hbm, zbuf)
    pltpu.sync_copy(zbuf, acc.at[pl.ds(s * RPT, RPT)])
    plsc.subcore_barrier()

    for k in range(NCHUNK):
        b = k % 2
        nb = (k + 1) % 2
        if k + 1 < NCHUNK:
            idx[nb][0].wait()
            idx[nb][1].wait()
            gathers[nb] = pltpu.async_copy(y_hbm.at[sbuf[nb]], rbuf[nb],
                                           gsems[nb])
        gathers[b].wait()
        pltpu.sync_copy(rbuf[b], acc.at[dbuf[b]], add=True)
        if k + 2 < NCHUNK:
            idx[b] = idx_start(k + 2)

    plsc.subcore_barrier()
    pltpu.sync_copy(acc.at[pl.ds(s * RPT, RPT)], out_hbm.at[c, pl.ds(s * RPT, RPT)])


def _tc_mm1_body(x_ref, w_ref, xw_ref):
    xw_ref[...] = jnp.dot(x_ref[...], w_ref[...],
                          preferred_element_type=jnp.float32)


def _dinv_packed(deg_ref):
    return lax.rsqrt(deg_ref[0, :NPR] + deg_ref[1, :NPR] + 1.0)


def _tc_scale_body(xw_ref, deg_ref, y_ref):
    y_ref[...] = xw_ref[...] * _dinv_packed(deg_ref)


def _tc_mid_body(acc_ref, y_ref, deg_ref, b_ref, wbd_ref, out_ref):
    dinv = _dinv_packed(deg_ref)
    h = dinv * (acc_ref[0, :NPR] + acc_ref[1, :NPR] + y_ref[...]) + b_ref[...]
    h = jnp.maximum(h, 0.0)
    out_ref[...] = jnp.dot(h, wbd_ref[...],
                           preferred_element_type=jnp.float32) * dinv


def _tc_final_body(acc_ref, y_ref, deg_ref, b_ref, batcht_ref, out_ref):
    dinv = _dinv_packed(deg_ref)
    z = dinv * (acc_ref[0, :NPR] + acc_ref[1, :NPR] + y_ref[...]) + b_ref[...]
    z = jnp.maximum(z, 0.0)
    gid = lax.broadcasted_iota(jnp.int32, (B, 1), 0)
    psum = jnp.zeros((B, F), jnp.float32)
    cnt = jnp.zeros((B, 1), jnp.float32)
    for u in range(PK):
        oh = (batcht_ref[u:u + 1, :] == gid).astype(jnp.float32)
        psum = psum + jnp.dot(oh, z[:, u * F:(u + 1) * F],
                              preferred_element_type=jnp.float32)
        cnt = cnt + jnp.sum(oh, axis=1, keepdims=True)
    pooled = psum / jnp.maximum(cnt, 1.0)
    m = jnp.max(pooled, axis=1, keepdims=True)
    ex = jnp.exp(pooled - m)
    lse = jnp.log(jnp.sum(ex, axis=1, keepdims=True))
    out_ref[...] = pooled - m - lse


_tc_mm1 = pl.pallas_call(
    _tc_mm1_body, out_shape=jax.ShapeDtypeStruct((NPR, 128), jnp.float32))
_tc_scale = pl.pallas_call(
    _tc_scale_body, out_shape=jax.ShapeDtypeStruct((NPR, 128), jnp.float32))
_tc_mid = pl.pallas_call(
    _tc_mid_body, out_shape=jax.ShapeDtypeStruct((NPR, 128), jnp.float32))
_tc_final = pl.pallas_call(
    _tc_final_body, out_shape=jax.ShapeDtypeStruct((B, F), jnp.float32))


def kernel(x, edge_index, batch, W1, b1, W2, b2):
    zeros_rp = jnp.zeros((RPT, F), jnp.float32)
    zeros1 = jnp.zeros((RPT,), jnp.float32)
    ones1 = jnp.ones((EPT,), jnp.float32)
    w1bd = jnp.kron(jnp.eye(PK, dtype=jnp.float32), W1)
    w2bd = jnp.kron(jnp.eye(PK, dtype=jnp.float32), W2)
    b1t = jnp.tile(b1, PK).reshape(1, 128)
    b2t = jnp.tile(b2, PK).reshape(1, 128)
    batch_t = batch.reshape(NPR, PK).T

    sc_degree = _build_sc_degree()
    sc_aggregate = _build_sc_aggregate()

    deg16 = sc_degree(edge_index, zeros1, ones1)
    degp = deg16.reshape(NC, NPR_PAD, 128)
    xwp = _tc_mm1(x.reshape(NPR, PK * D), w1bd)
    y1p = _tc_scale(xwp, degp)
    acc1 = sc_aggregate(y1p.reshape(N, F), edge_index, zeros_rp)
    y2p = _tc_mid(acc1.reshape(NC, NPR_PAD, 128), y1p, degp, b1t, w2bd)
    acc2 = sc_aggregate(y2p.reshape(N, F), edge_index, zeros_rp)
    out = _tc_final(acc2.reshape(NC, NPR_PAD, 128), y2p, degp, b2t, batch_t)
    return out


# speedup vs baseline: 1.1913x; 1.0039x over previous
import functools

import jax
import jax.numpy as jnp
from jax import lax
from jax.experimental import pallas as pl
from jax.experimental.pallas import tpu as pltpu
from jax.experimental.pallas import tpu_sc as plsc

N = 10000
E = 320000
D = 128
F = 16
B = 64

NC = 2
NS = 16
NW = NC * NS
N_PAD = 10240
RPT = N_PAD // NS
EPT = E // NW
CHUNK = 2000
NCHUNK = EPT // CHUNK

PK = 128 // F
NPR = N // PK
NPR_PAD = N_PAD // PK


def _mesh():
    return plsc.VectorSubcoreMesh(
        core_axis_name="c", subcore_axis_name="s", num_cores=NC, num_subcores=NS
    )


@functools.cache
def _build_sc_degree():
    return pl.kernel(
        _sc_degree_body,
        out_type=jax.ShapeDtypeStruct((NC, N_PAD, F), jnp.float32),
        mesh=_mesh(),
        compiler_params=pltpu.CompilerParams(use_tc_tiling_on_sc=False),
        scratch_types=[
            pltpu.VMEM((EPT,), jnp.int32),
            pltpu.VMEM((EPT,), jnp.float32),
            pltpu.VMEM((RPT,), jnp.float32),
            pltpu.VMEM((RPT, F), jnp.float32),
            pltpu.VMEM_SHARED((N_PAD,), jnp.float32),
            pltpu.SemaphoreType.DMA,
        ],
    )


def _sc_degree_body(ei_hbm, zeros1_hbm, ones1_hbm, out_hbm, didx,
                    ones, dloc, zbuf, acc, sem):
    c = lax.axis_index("c")
    s = lax.axis_index("s")
    base = (s * NC + c) * EPT
    idxcp = pltpu.async_copy(ei_hbm.at[1, pl.ds(base, EPT)], didx, sem)
    pltpu.sync_copy(zeros1_hbm, dloc)
    pltpu.sync_copy(ones1_hbm, ones)
    pltpu.sync_copy(dloc, acc.at[pl.ds(s * RPT, RPT)])
    plsc.subcore_barrier()

    idxcp.wait()
    pltpu.sync_copy(ones, acc.at[didx], add=True)

    plsc.subcore_barrier()
    pltpu.sync_copy(acc.at[pl.ds(s * RPT, RPT)], dloc)

    def expand(g, carry):
        v = dloc[pl.ds(g * 16, 16)]
        for u in range(16):
            zbuf[g * 16 + u, :] = jnp.full((F,), v[u], jnp.float32)
        return carry

    lax.fori_loop(0, RPT // 16, expand, 0)
    pltpu.sync_copy(zbuf, out_hbm.at[c, pl.ds(s * RPT, RPT)])


@functools.cache
def _build_sc_aggregate():
    return pl.kernel(
        _sc_aggregate_body,
        out_type=jax.ShapeDtypeStruct((NC, N_PAD, F), jnp.float32),
        mesh=_mesh(),
        compiler_params=pltpu.CompilerParams(use_tc_tiling_on_sc=False),
        scratch_types=(
            [pltpu.VMEM((CHUNK,), jnp.int32)] * 4 +
            [pltpu.VMEM((CHUNK,), jnp.int32)] * 4 +
            [pltpu.VMEM((CHUNK, F), jnp.float32)] * 2 +
            [pltpu.VMEM((RPT, F), jnp.float32),
             pltpu.VMEM_SHARED((N_PAD, F), jnp.float32)] +
            [pltpu.SemaphoreType.DMA] * 8
        ),
    )


def _sc_aggregate_body(y_hbm, ei_hbm, zeros_hbm, out_hbm,
                       si0, si1, si2, si3, di0, di1, di2, di3, rows0, rows1,
                       zbuf, acc, gsem0, gsem1, isem0, isem1, isem2, isem3,
                       ssem0, ssem1):
    c = lax.axis_index("c")
    s = lax.axis_index("s")
    base = (s * NC + c) * EPT
    sbuf = (si0, si1, si2, si3)
    dbuf = (di0, di1, di2, di3)
    rbuf = (rows0, rows1)
    gsems = (gsem0, gsem1)
    isems = (isem0, isem1, isem2, isem3)
    ssems = (ssem0, ssem1)

    def idx_start(k):
        j = k % 4
        e0 = base + k * CHUNK
        cs = pltpu.async_copy(ei_hbm.at[0, pl.ds(e0, CHUNK)], sbuf[j],
                              isems[j])
        cd = pltpu.async_copy(ei_hbm.at[1, pl.ds(e0, CHUNK)], dbuf[j],
                              isems[j])
        return (cs, cd)

    idx = [None] * 4
    gathers = [None, None]
    scats = [None] * NCHUNK
    idx[0] = idx_start(0)
    if NCHUNK > 1:
        idx[1] = idx_start(1)
    idx[0][0].wait()
    idx[0][1].wait()
    gathers[0] = pltpu.async_copy(y_hbm.at[sbuf[0]], rows0, gsem0)

    pltpu.sync_copy(zeros_hbm, zbuf)
    pltpu.sync_copy(zbuf, acc.at[pl.ds(s * RPT, RPT)])
    plsc.subcore_barrier()

    for k in range(NCHUNK):
        b = k % 2
        nb = (k + 1) % 2
        if k + 1 < NCHUNK:
            idx[(k + 1) % 4][0].wait()
            idx[(k + 1) % 4][1].wait()
            if k >= 1:
                scats[k - 1].wait()
            gathers[nb] = pltpu.async_copy(y_hbm.at[sbuf[(k + 1) % 4]],
                                           rbuf[nb], gsems[nb])
        gathers[b].wait()
        scats[k] = pltpu.async_copy(rbuf[b], acc.at[dbuf[k % 4]], ssems[b],
                                    add=True)
        if k + 2 < NCHUNK:
            idx[(k + 2) % 4] = idx_start(k + 2)

    if NCHUNK >= 2:
        scats[NCHUNK - 2].wait()
    scats[NCHUNK - 1].wait()
    plsc.subcore_barrier()
    pltpu.sync_copy(acc.at[pl.ds(s * RPT, RPT)], out_hbm.at[c, pl.ds(s * RPT, RPT)])


def _tc_mm1_body(x_ref, w_ref, xw_ref):
    xw_ref[...] = jnp.dot(x_ref[...], w_ref[...],
                          preferred_element_type=jnp.float32)


def _dinv_packed(deg_ref):
    return lax.rsqrt(deg_ref[0, :NPR] + deg_ref[1, :NPR] + 1.0)


def _tc_scale_body(xw_ref, deg_ref, y_ref):
    y_ref[...] = xw_ref[...] * _dinv_packed(deg_ref)


def _tc_mid_body(acc_ref, y_ref, deg_ref, b_ref, wbd_ref, out_ref):
    dinv = _dinv_packed(deg_ref)
    h = dinv * (acc_ref[0, :NPR] + acc_ref[1, :NPR] + y_ref[...]) + b_ref[...]
    h = jnp.maximum(h, 0.0)
    out_ref[...] = jnp.dot(h, wbd_ref[...],
                           preferred_element_type=jnp.float32) * dinv


def _tc_final_body(acc_ref, y_ref, deg_ref, b_ref, batcht_ref, out_ref):
    dinv = _dinv_packed(deg_ref)
    z = dinv * (acc_ref[0, :NPR] + acc_ref[1, :NPR] + y_ref[...]) + b_ref[...]
    z = jnp.maximum(z, 0.0)
    gid = lax.broadcasted_iota(jnp.int32, (B, 1), 0)
    psum = jnp.zeros((B, F), jnp.float32)
    cnt = jnp.zeros((B, 1), jnp.float32)
    for u in range(PK):
        oh = (batcht_ref[u:u + 1, :] == gid).astype(jnp.float32)
        psum = psum + jnp.dot(oh, z[:, u * F:(u + 1) * F],
                              preferred_element_type=jnp.float32)
        cnt = cnt + jnp.sum(oh, axis=1, keepdims=True)
    pooled = psum / jnp.maximum(cnt, 1.0)
    m = jnp.max(pooled, axis=1, keepdims=True)
    ex = jnp.exp(pooled - m)
    lse = jnp.log(jnp.sum(ex, axis=1, keepdims=True))
    out_ref[...] = pooled - m - lse


_tc_mm1 = pl.pallas_call(
    _tc_mm1_body, out_shape=jax.ShapeDtypeStruct((NPR, 128), jnp.float32))
_tc_scale = pl.pallas_call(
    _tc_scale_body, out_shape=jax.ShapeDtypeStruct((NPR, 128), jnp.float32))
_tc_mid = pl.pallas_call(
    _tc_mid_body, out_shape=jax.ShapeDtypeStruct((NPR, 128), jnp.float32))
_tc_final = pl.pallas_call(
    _tc_final_body, out_shape=jax.ShapeDtypeStruct((B, F), jnp.float32))


def kernel(x, edge_index, batch, W1, b1, W2, b2):
    zeros_rp = jnp.zeros((RPT, F), jnp.float32)
    zeros1 = jnp.zeros((RPT,), jnp.float32)
    ones1 = jnp.ones((EPT,), jnp.float32)
    w1bd = jnp.kron(jnp.eye(PK, dtype=jnp.float32), W1)
    w2bd = jnp.kron(jnp.eye(PK, dtype=jnp.float32), W2)
    b1t = jnp.tile(b1, PK).reshape(1, 128)
    b2t = jnp.tile(b2, PK).reshape(1, 128)
    batch_t = batch.reshape(NPR, PK).T

    sc_degree = _build_sc_degree()
    sc_aggregate = _build_sc_aggregate()

    deg16 = sc_degree(edge_index, zeros1, ones1)
    degp = deg16.reshape(NC, NPR_PAD, 128)
    xwp = _tc_mm1(x.reshape(NPR, PK * D), w1bd)
    y1p = _tc_scale(xwp, degp)
    acc1 = sc_aggregate(y1p.reshape(N, F), edge_index, zeros_rp)
    y2p = _tc_mid(acc1.reshape(NC, NPR_PAD, 128), y1p, degp, b1t, w2bd)
    acc2 = sc_aggregate(y2p.reshape(N, F), edge_index, zeros_rp)
    out = _tc_final(acc2.reshape(NC, NPR_PAD, 128), y2p, degp, b2t, batch_t)
    return out
